# baseline (speedup 1.0000x reference)
import functools

import jax
import jax.numpy as jnp
from jax import lax
from jax.experimental import pallas as pl
from jax.experimental.pallas import tpu as pltpu
from jax.experimental.pallas import tpu_sc as plsc

_BM = 1024
_CH = 128


def _row_index_kernel(n, a_ref, out_ref):
    i = pl.program_id(0)
    w = a_ref[...]
    nz = (w != 0).astype(jnp.int32)
    p = lax.broadcasted_iota(jnp.int32, w.shape, 1)
    wpos = jnp.sum(nz * p, axis=1)
    wval = jnp.max(w, axis=1)
    bits = lax.bitcast_convert_type(wval.astype(jnp.float32), jnp.int32)
    k = ((bits >> 23) - 127) >> 3
    base = i * _BM // n * n
    out_ref[0, 0, :] = (wpos << 2) + k + base


def _extract_indices(a_w, n):
    rows, wpr = a_w.shape
    nb = rows // _BM
    out = pl.pallas_call(
        functools.partial(_row_index_kernel, n),
        grid=(nb,),
        in_specs=[pl.BlockSpec((_BM, wpr), lambda i: (i, 0))],
        out_specs=pl.BlockSpec((1, 1, _BM), lambda i: (i, 0, 0)),
        out_shape=jax.ShapeDtypeStruct((nb, 1, _BM), jnp.int32),
    )(a_w)
    return out.reshape(rows)


def _sc_gather(table, idx2d):
    rows, d = table.shape
    info = plsc.get_sparse_core_info()
    nc, ns = info.num_cores, info.num_subcores
    nw = nc * ns
    per_w = rows // nw
    k = per_w // _CH
    mesh = plsc.VectorSubcoreMesh(core_axis_name="c", subcore_axis_name="s")

    @functools.partial(
        pl.kernel,
        mesh=mesh,
        out_type=jax.ShapeDtypeStruct((rows, d), jnp.float32),
        scratch_types=[
            pltpu.VMEM((k, _CH), jnp.int32),
            pltpu.VMEM((per_w, d), jnp.float32),
            pltpu.SemaphoreType.DMA,
        ],
    )
    def run(tab_hbm, idx_hbm, out_hbm, idx_v, rows_v, sem):
        wid = lax.axis_index("s") * nc + lax.axis_index("c")
        base = wid * per_w
        pltpu.sync_copy(idx_hbm.at[pl.ds(wid * k, k)], idx_v)
        copies = [
            pltpu.async_copy(tab_hbm.at[idx_v.at[j]],
                             rows_v.at[pl.ds(j * _CH, _CH)], sem)
            for j in range(k)
        ]
        for c in copies:
            c.wait()
        pltpu.sync_copy(rows_v, out_hbm.at[pl.ds(base, per_w)])

    return run(table, idx2d)


def kernel(entities, assignments):
    b, n, d = entities.shape
    a_w = assignments.view(jnp.int32)
    flat_idx = _extract_indices(a_w.reshape(b * n, n // 4), n)
    out = _sc_gather(entities.reshape(b * n, d), flat_idx.reshape(-1, _CH))
    return out.reshape(b, n, d)


# speedup vs baseline: 3.6072x; 3.6072x over previous
import functools

import jax
import jax.numpy as jnp
from jax import lax
from jax.experimental import pallas as pl
from jax.experimental.pallas import tpu as pltpu
from jax.experimental.pallas import tpu_sc as plsc

_BM = 1024
_CH = 128


def _row_index_kernel(n, a_ref, out_ref):
    i = pl.program_id(0)
    a = a_ref[...]
    bm = a.shape[0]
    lane = lax.broadcasted_iota(jnp.int16, (bm, 128), 1)
    acc = jnp.zeros((bm, 128), jnp.int16)
    for v in range(n // 128):
        av = lax.slice_in_dim(a, v * 128, (v + 1) * 128, axis=1)
        col = lane + jnp.int16(128 * v)
        acc = acc + av.astype(jnp.int16) * col
    idx = jnp.sum(acc.astype(jnp.int32), axis=1)
    base = i * _BM // n * n
    out_ref[0, 0, :] = idx + base


def _extract_indices(a_i8, n):
    rows, _ = a_i8.shape
    nb = rows // _BM
    out = pl.pallas_call(
        functools.partial(_row_index_kernel, n),
        grid=(nb,),
        in_specs=[pl.BlockSpec((_BM, n), lambda i: (i, 0))],
        out_specs=pl.BlockSpec((1, 1, _BM), lambda i: (i, 0, 0)),
        out_shape=jax.ShapeDtypeStruct((nb, 1, _BM), jnp.int32),
    )(a_i8)
    return out.reshape(rows)


def _sc_gather(table, idx2d):
    rows, d = table.shape
    info = plsc.get_sparse_core_info()
    nc, ns = info.num_cores, info.num_subcores
    nw = nc * ns
    per_w = rows // nw
    k = per_w // _CH
    mesh = plsc.VectorSubcoreMesh(core_axis_name="c", subcore_axis_name="s")

    @functools.partial(
        pl.kernel,
        mesh=mesh,
        out_type=jax.ShapeDtypeStruct((rows, d), jnp.float32),
        scratch_types=[
            pltpu.VMEM((k, _CH), jnp.int32),
            pltpu.VMEM((per_w, d), jnp.float32),
            pltpu.SemaphoreType.DMA,
        ],
    )
    def run(tab_hbm, idx_hbm, out_hbm, idx_v, rows_v, sem):
        wid = lax.axis_index("s") * nc + lax.axis_index("c")
        base = wid * per_w
        pltpu.sync_copy(idx_hbm.at[pl.ds(wid * k, k)], idx_v)
        copies = [
            pltpu.async_copy(tab_hbm.at[idx_v.at[j]],
                             rows_v.at[pl.ds(j * _CH, _CH)], sem)
            for j in range(k)
        ]
        for c in copies:
            c.wait()
        pltpu.sync_copy(rows_v, out_hbm.at[pl.ds(base, per_w)])

    return run(table, idx2d)


def kernel(entities, assignments):
    b, n, d = entities.shape
    a_i8 = assignments.view(jnp.int8)
    flat_idx = _extract_indices(a_i8.reshape(b * n, n), n)
    out = _sc_gather(entities.reshape(b * n, d), flat_idx.reshape(-1, _CH))
    return out.reshape(b, n, d)
